# baseline (speedup 1.0000x reference)
import functools

import jax
import jax.numpy as jnp
from jax import lax
from jax.experimental import pallas as pl
from jax.experimental.pallas import tpu as pltpu
from jax.experimental.pallas import tpu_sc as plsc

_VOCAB = 1000000
_DIM = 64
_HIDDEN = 2000
_LABELS = 1000
_B = 4096
_L = 200
_EPS = 1e-5

_NC = 2
_NS = 16
_NW = _NC * _NS
_BPW = _B // _NW
_FIRE = 8


_HL = _L // 2
_PD = 128
_NBUF = 4
_SA = 128
_SB = _L - _SA
_WTOK = _BPW * _L


def _sc_gather_pool(content_w, emb_flat):
  mesh = plsc.VectorSubcoreMesh(
      core_axis_name="c", subcore_axis_name="s", num_cores=_NC,
      num_subcores=_NS)

  @functools.partial(
      pl.kernel,
      out_type=jax.ShapeDtypeStruct((_B, _DIM), jnp.float32),
      mesh=mesh,
      compiler_params=pltpu.CompilerParams(use_tc_tiling_on_sc=False),
      scratch_types=[
          pltpu.VMEM((_WTOK,), jnp.int32),
          pltpu.VMEM((_NBUF, _L, _DIM), jnp.float32),
          pltpu.VMEM((_BPW, _DIM), jnp.float32),
          tuple(pltpu.SemaphoreType.DMA for _ in range(_NBUF)),
      ],
  )
  def body(content_hbm, emb_hbm, out_hbm, idx_v, buf_v, stage_v, sems):
    wid = lax.axis_index("s") * _NC + lax.axis_index("c")
    base = wid * _BPW
    pltpu.sync_copy(content_hbm.at[wid], idx_v)

    def remap(c, carry):
      off = c * 16
      t = idx_v[pl.ds(off, 16)]
      std = (t & -_VB) + 2 * (t & (_VB // 2 - 1)) + (
          lax.shift_right_logical(t, (_VB // 2).bit_length() - 1) & 1)
      idx_v[pl.ds(off, 16)] = std
      return carry

    lax.fori_loop(0, _WTOK // 16, remap, 0, unroll=False)

    def fire(e, slot):
      pltpu.async_copy(
          emb_hbm.at[idx_v.at[pl.ds(e * _L, _SA)]],
          buf_v.at[slot, pl.ds(0, _SA)], sems[slot])
      pltpu.async_copy(
          emb_hbm.at[idx_v.at[pl.ds(e * _L + _SA, _SB)]],
          buf_v.at[slot, pl.ds(_SA, _SB)], sems[slot])

    def drain(e, slot):
      pltpu.make_async_copy(
          emb_hbm.at[idx_v.at[pl.ds(e * _L, _SA)]],
          buf_v.at[slot, pl.ds(0, _SA)], sems[slot]).wait()
      pltpu.make_async_copy(
          emb_hbm.at[idx_v.at[pl.ds(e * _L + _SA, _SB)]],
          buf_v.at[slot, pl.ds(_SA, _SB)], sems[slot]).wait()

    def accum(e, slot):
      def jchunk(jc, accs):
        out = list(accs)
        for jj in range(8):
          j = jc * 8 + jj
          for k in range(4):
            out[k] = out[k] + buf_v[slot, j, pl.ds(k * 16, 16)]
        return tuple(out)

      accs = lax.fori_loop(
          0, _L // 8, jchunk,
          tuple(jnp.zeros((16,), jnp.float32) for _ in range(4)),
          unroll=False)
      for k in range(4):
        stage_v[e, pl.ds(k * 16, 16)] = accs[k]

    for s in range(_NBUF):
      fire(s, s)

    def step(t, carry):
      for u in range(_NBUF):
        e = _NBUF * t + u
        drain(e, u)
        accum(e, u)

        @pl.when(e + _NBUF < _BPW)
        def _():
          fire(e + _NBUF, u)

      return carry

    n_full = _BPW // _NBUF
    lax.fori_loop(0, n_full, step, 0, unroll=False)
    for e in range(n_full * _NBUF, _BPW):
      drain(e, e % _NBUF)
      accum(e, e % _NBUF)

    pltpu.sync_copy(stage_v, out_hbm.at[pl.ds(base, _BPW)])

  return body(content_w, emb_flat)


_VB = 8192
_NVB = pl.cdiv(_VOCAB, _VB)
_PROWS = _NVB * (_VB // 2)
_TAIL0 = (_NVB - 1) * _VB
_TAILOFF = 2 * (_VOCAB - _VB) + _VB - (_NVB - 1) * _VB - 1


def _transpose_body(in_ref, out_ref):
  x = in_ref[...]
  h, q = _VB // 2, _VB // 4
  ri = lax.broadcasted_iota(jnp.int32, (_DIM, _DIM), 0)
  ci = lax.broadcasted_iota(jnp.int32, (_DIM, _DIM), 1)
  eye = (ri == ci).astype(jnp.float32)

  def dot_t(xs):
    return lax.dot_general(xs, eye, (((0,), (0,)), ((), ())),
                           preferred_element_type=jnp.float32)

  out_ref[pl.ds(0, q), :] = jnp.concatenate(
      [x[:, :q].T, x[:, h:h + q].T], axis=1)
  out_ref[pl.ds(q, q), :] = jnp.concatenate(
      [dot_t(x[:, q:h]), dot_t(x[:, h + q:])], axis=1)


def _tc_table_transpose(emb_t):
  return pl.pallas_call(
      _transpose_body,
      grid=(_NVB,),
      in_specs=[pl.BlockSpec((_DIM, _VB), lambda i: (0, i))],
      out_specs=pl.BlockSpec((_VB // 2, _PD), lambda i: (i, 0)),
      out_shape=jax.ShapeDtypeStruct((_PROWS, _PD), jnp.float32),
      compiler_params=pltpu.CompilerParams(fuse_transposed_lhs_in_matmul=True),
  )(emb_t)


_BB = 512


def _mlp_body(csum_ref, w1_ref, gamma_ref, beta_ref, w2_ref, b2_ref, out_ref,
              s_ref, m_ref):
  i = pl.program_id(0)

  @pl.when(i == 0)
  def _():
    c = csum_ref[...] * (1.0 / _L)
    m = jnp.mean(c, axis=0)
    g = lax.dot_general(c, c, (((0,), (0,)), ((), ())),
                        preferred_element_type=jnp.float32) / _B
    cov = g - m[:, None] * m[None, :]
    a = lax.dot_general(w1_ref[...], cov, (((1,), (0,)), ((), ())),
                        preferred_element_type=jnp.float32)
    var = jnp.sum(a * w1_ref[...], axis=1)
    s_ref[...] = (gamma_ref[...] * lax.rsqrt(var + _EPS)[None, :])
    m_ref[...] = m[None, :]

  blk = csum_ref[pl.ds(i * _BB, _BB), :] * (1.0 / _L) - m_ref[...]
  h = lax.dot_general(blk, w1_ref[...], (((1,), (1,)), ((), ())),
                      preferred_element_type=jnp.float32)
  r = jnp.maximum(h * s_ref[...] + beta_ref[...], 0.0)
  out_ref[...] = lax.dot_general(r, w2_ref[...], (((1,), (1,)), ((), ())),
                                 preferred_element_type=jnp.float32) + b2_ref[...]


def _tc_mlp(csum, w1, gamma, beta, w2, b2):
  grid = (_B // _BB,)
  full = lambda shape: pl.BlockSpec(shape, lambda i: (0, 0))
  return pl.pallas_call(
      _mlp_body,
      grid=grid,
      in_specs=[
          full((_B, _DIM)),
          full((_HIDDEN, _DIM)),
          full((1, _HIDDEN)),
          full((1, _HIDDEN)),
          full((_LABELS, _HIDDEN)),
          full((1, _LABELS)),
      ],
      out_specs=pl.BlockSpec((_BB, _LABELS), lambda i: (i, 0)),
      out_shape=jax.ShapeDtypeStruct((_B, _LABELS), jnp.float32),
      scratch_shapes=[
          pltpu.VMEM((1, _HIDDEN), jnp.float32),
          pltpu.VMEM((1, _DIM), jnp.float32),
      ],
  )(csum, w1, gamma.reshape(1, _HIDDEN), beta.reshape(1, _HIDDEN), w2,
    b2.reshape(1, _LABELS))


def kernel(content, emb, W1, b1, gamma, beta, W2, b2):
  del b1
  content_w = content.astype(jnp.int32).reshape(_NW, _WTOK)
  emb128 = _tc_table_transpose(emb.T)
  emb_flat = emb128.reshape(2 * _PROWS, _DIM)
  csum = _sc_gather_pool(content_w, emb_flat)
  return _tc_mlp(csum, W1, gamma, beta, W2, b2)


# speedup vs baseline: 1.0033x; 1.0033x over previous
import functools

import jax
import jax.numpy as jnp
from jax import lax
from jax.experimental import pallas as pl
from jax.experimental.pallas import tpu as pltpu
from jax.experimental.pallas import tpu_sc as plsc

_VOCAB = 1000000
_DIM = 64
_HIDDEN = 2000
_LABELS = 1000
_B = 4096
_L = 200
_EPS = 1e-5

_NC = 2
_NS = 16
_NW = _NC * _NS
_BPW = _B // _NW

_PD = 128
_NBUF = 4
_SA = 128
_SB = _L - _SA
_WTOK = _BPW * _L


def _sc_gather_pool(content_w, emb_flat):
  mesh = plsc.VectorSubcoreMesh(
      core_axis_name="c", subcore_axis_name="s", num_cores=_NC,
      num_subcores=_NS)

  @functools.partial(
      pl.kernel,
      out_type=jax.ShapeDtypeStruct((_B, _DIM), jnp.float32),
      mesh=mesh,
      compiler_params=pltpu.CompilerParams(use_tc_tiling_on_sc=False),
      scratch_types=[
          pltpu.VMEM((_WTOK,), jnp.int32),
          pltpu.VMEM((_NBUF, _L, _DIM), jnp.float32),
          pltpu.VMEM((_BPW, _DIM), jnp.float32),
          tuple(pltpu.SemaphoreType.DMA for _ in range(_NBUF)),
      ],
  )
  def body(content_hbm, emb_hbm, out_hbm, idx_v, buf_v, stage_v, sems):
    wid = lax.axis_index("s") * _NC + lax.axis_index("c")
    base = wid * _BPW
    pltpu.sync_copy(content_hbm.at[wid], idx_v)

    def remap(c, carry):
      off = c * 16
      t = idx_v[pl.ds(off, 16)]
      std = (t & -_VB) + 2 * (t & (_VB // 2 - 1)) + (
          lax.shift_right_logical(t, (_VB // 2).bit_length() - 1) & 1)
      idx_v[pl.ds(off, 16)] = std
      return carry

    lax.fori_loop(0, _WTOK // 16, remap, 0, unroll=False)

    def fire(e, slot):
      pltpu.async_copy(
          emb_hbm.at[idx_v.at[pl.ds(e * _L, _SA)]],
          buf_v.at[slot, pl.ds(0, _SA)], sems[slot])
      pltpu.async_copy(
          emb_hbm.at[idx_v.at[pl.ds(e * _L + _SA, _SB)]],
          buf_v.at[slot, pl.ds(_SA, _SB)], sems[slot])

    def drain(e, slot):
      pltpu.make_async_copy(
          emb_hbm.at[idx_v.at[pl.ds(e * _L, _SA)]],
          buf_v.at[slot, pl.ds(0, _SA)], sems[slot]).wait()
      pltpu.make_async_copy(
          emb_hbm.at[idx_v.at[pl.ds(e * _L + _SA, _SB)]],
          buf_v.at[slot, pl.ds(_SA, _SB)], sems[slot]).wait()

    def accum(e, slot):
      def jchunk(jc, accs):
        out = list(accs)
        for jj in range(8):
          j = jc * 8 + jj
          for k in range(4):
            out[k] = out[k] + buf_v[slot, j, pl.ds(k * 16, 16)]
        return tuple(out)

      accs = lax.fori_loop(
          0, _L // 8, jchunk,
          tuple(jnp.zeros((16,), jnp.float32) for _ in range(4)),
          unroll=False)
      for k in range(4):
        stage_v[e, pl.ds(k * 16, 16)] = accs[k]

    for s in range(_NBUF):
      fire(s, s)

    def step(t, carry):
      for u in range(_NBUF):
        e = _NBUF * t + u
        drain(e, u)
        accum(e, u)

        @pl.when(e + _NBUF < _BPW)
        def _():
          fire(e + _NBUF, u)

      return carry

    n_full = _BPW // _NBUF
    lax.fori_loop(0, n_full, step, 0, unroll=False)
    for e in range(n_full * _NBUF, _BPW):
      drain(e, e % _NBUF)
      accum(e, e % _NBUF)

    pltpu.sync_copy(stage_v, out_hbm.at[pl.ds(base, _BPW)])

  return body(content_w, emb_flat)


_VB = 8192
_NVB = pl.cdiv(_VOCAB, _VB)
_PROWS = _NVB * (_VB // 2)


def _transpose_body(in_ref, out_ref):
  x = in_ref[...]
  out_ref[...] = jnp.concatenate(
      [x[:, :_VB // 2].T, x[:, _VB // 2:].T], axis=1)


def _tc_table_transpose(emb_t):
  return pl.pallas_call(
      _transpose_body,
      grid=(_NVB,),
      in_specs=[pl.BlockSpec((_DIM, _VB), lambda i: (0, i))],
      out_specs=pl.BlockSpec((_VB // 2, _PD), lambda i: (i, 0)),
      out_shape=jax.ShapeDtypeStruct((_PROWS, _PD), jnp.float32),
  )(emb_t)


_BB = 512


def _mlp_body(csum_ref, w1_ref, gamma_ref, beta_ref, w2_ref, b2_ref, out_ref,
              s_ref, m_ref):
  i = pl.program_id(0)

  @pl.when(i == 0)
  def _():
    c = csum_ref[...] * (1.0 / _L)
    m = jnp.mean(c, axis=0)
    g = lax.dot_general(c, c, (((0,), (0,)), ((), ())),
                        preferred_element_type=jnp.float32) / _B
    cov = g - m[:, None] * m[None, :]
    a = lax.dot_general(w1_ref[...], cov, (((1,), (0,)), ((), ())),
                        preferred_element_type=jnp.float32)
    var = jnp.sum(a * w1_ref[...], axis=1)
    s_ref[...] = (gamma_ref[...] * lax.rsqrt(var + _EPS)[None, :])
    m_ref[...] = m[None, :]

  blk = csum_ref[pl.ds(i * _BB, _BB), :] * (1.0 / _L) - m_ref[...]
  h = lax.dot_general(blk, w1_ref[...], (((1,), (1,)), ((), ())),
                      preferred_element_type=jnp.float32)
  r = jnp.maximum(h * s_ref[...] + beta_ref[...], 0.0)
  out_ref[...] = lax.dot_general(r, w2_ref[...], (((1,), (1,)), ((), ())),
                                 preferred_element_type=jnp.float32) + b2_ref[...]


def _tc_mlp(csum, w1, gamma, beta, w2, b2):
  grid = (_B // _BB,)
  full = lambda shape: pl.BlockSpec(shape, lambda i: (0, 0))
  return pl.pallas_call(
      _mlp_body,
      grid=grid,
      in_specs=[
          full((_B, _DIM)),
          full((_HIDDEN, _DIM)),
          full((1, _HIDDEN)),
          full((1, _HIDDEN)),
          full((_LABELS, _HIDDEN)),
          full((1, _LABELS)),
      ],
      out_specs=pl.BlockSpec((_BB, _LABELS), lambda i: (i, 0)),
      out_shape=jax.ShapeDtypeStruct((_B, _LABELS), jnp.float32),
      scratch_shapes=[
          pltpu.VMEM((1, _HIDDEN), jnp.float32),
          pltpu.VMEM((1, _DIM), jnp.float32),
      ],
  )(csum, w1, gamma.reshape(1, _HIDDEN), beta.reshape(1, _HIDDEN), w2,
    b2.reshape(1, _LABELS))


def kernel(content, emb, W1, b1, gamma, beta, W2, b2):
  del b1
  content_w = content.astype(jnp.int32).reshape(_NW, _WTOK)
  emb128 = _tc_table_transpose(emb.T)
  emb_flat = emb128.reshape(2 * _PROWS, _DIM)
  csum = _sc_gather_pool(content_w, emb_flat)
  return _tc_mlp(csum, W1, gamma, beta, W2, b2)
